# Initial kernel scaffold; baseline (speedup 1.0000x reference)
import jax
import jax.numpy as jnp
from jax.experimental import pallas as pl


def kernel(token_ids, sent_lengths, embeds_weight):
    raise NotImplementedError("write your pallas kernel here")


# speedup vs baseline: 10.2473x; 10.2473x over previous
import functools

import jax
import jax.numpy as jnp
from jax import lax
from jax.experimental import pallas as pl
from jax.experimental.pallas import tpu as pltpu
from jax.experimental.pallas import tpu_sc as plsc

L = 16


def _mean_encoder_sc(token_ids, sent_lengths, embeds_weight, *, G=32):
    TOTAL = token_ids.shape[0]
    B = sent_lengths.shape[0]
    E = embeds_weight.shape[1]

    info = plsc.get_sparse_core_info()
    NC, NS = info.num_cores, info.num_subcores
    NW = NC * NS
    TPW = TOTAL // NW
    WPS = NW // B
    NB = TPW // G
    EC = E // L
    HALF = E // WPS

    mesh = plsc.VectorSubcoreMesh(core_axis_name="c", subcore_axis_name="s")

    @functools.partial(
        pl.kernel,
        out_type=jax.ShapeDtypeStruct((B, E), jnp.float32),
        mesh=mesh,
        scratch_types=[
            pltpu.VMEM((TPW,), jnp.int32),
            pltpu.VMEM((G, E), jnp.float32),
            pltpu.VMEM((G, E), jnp.float32),
            pltpu.VMEM((E,), jnp.float32),
            pltpu.VMEM((1, E), jnp.float32),
            pltpu.VMEM((HALF,), jnp.float32),
            pltpu.VMEM((HALF,), jnp.float32),
            pltpu.VMEM((L,), jnp.float32),
            pltpu.VMEM_SHARED((NS, E), jnp.float32),
            pltpu.SemaphoreType.DMA,
            pltpu.SemaphoreType.DMA,
            pltpu.SemaphoreType.DMA,
        ],
    )
    def k(tok_hbm, rec_hbm, w_hbm, out_hbm, idx_v, buf0, buf1, acc_v,
          row1_v, part_v, outrow_v, rec_v, shared, sem0, sem1, sem2):
        c = lax.axis_index("c")
        s = lax.axis_index("s")
        wid = c * NS + s
        base = wid * TPW
        sent = wid // WPS
        halfsel = wid % WPS
        col0 = halfsel * HALF

        pltpu.sync_copy(tok_hbm.at[pl.ds(base, TPW)], idx_v)
        pltpu.sync_copy(rec_hbm.at[wid], rec_v)

        bufs = (buf0, buf1)
        sems = (sem0, sem1)

        def start(g, buf, sem):
            pltpu.make_async_copy(
                w_hbm.at[idx_v.at[pl.ds(g * G, G)]], buf, sem).start()

        def wait(buf, sem):
            pltpu.make_async_copy(
                w_hbm.at[idx_v.at[pl.ds(0, G)]], buf, sem).wait()

        zero = jnp.zeros((L,), jnp.float32)

        def zbody(j, carry):
            acc_v[pl.ds(j * L, L)] = zero
            return carry

        lax.fori_loop(0, EC, zbody, 0)

        start(0, buf0, sem0)
        pltpu.make_async_copy(
            w_hbm.at[idx_v.at[pl.ds(0, 1)]], row1_v, sem2).start()

        def accum(buf):
            def abody(j, carry):
                dsl = pl.ds(j * L, L)
                v = acc_v[dsl]
                for r in range(G):
                    v = v + buf[r, dsl]
                acc_v[dsl] = v
                return carry
            lax.fori_loop(0, EC, abody, 0)

        def pair_body(i, carry):
            for half in range(2):
                g = i * 2 + half
                p = half
                nxt = g + 1

                @pl.when(nxt < NB)
                def _():
                    start(nxt, bufs[1 - p], sems[1 - p])

                wait(bufs[p], sems[p])
                accum(bufs[p])
            return carry

        lax.fori_loop(0, NB // 2, pair_body, 0)

        pltpu.make_async_copy(
            w_hbm.at[idx_v.at[pl.ds(0, 1)]], row1_v, sem2).wait()
        exm = jnp.where(halfsel == 0, 1.0, 0.0).astype(jnp.float32)

        def sbody(j, carry):
            dsl = pl.ds(j * L, L)
            acc_v[dsl] = acc_v[dsl] - row1_v[0, dsl] * exm
            return carry

        lax.fori_loop(0, EC, sbody, 0)

        pltpu.sync_copy(acc_v, shared.at[s])
        plsc.subcore_barrier()
        partner = jnp.bitwise_xor(s, 1)
        pltpu.sync_copy(shared.at[partner, pl.ds(col0, HALF)], part_v)

        rec = rec_v[...]

        def fbody(j, carry):
            dsl = pl.ds(col0 + j * L, L)
            dso = pl.ds(j * L, L)
            outrow_v[dso] = (acc_v[dsl] + part_v[dso]) * rec
            return carry

        lax.fori_loop(0, HALF // L, fbody, 0)

        pltpu.sync_copy(outrow_v, out_hbm.at[sent, pl.ds(col0, HALF)])

    recips = (1.0 / sent_lengths.astype(jnp.float32))
    recips = jnp.repeat(recips, WPS)
    recips = jnp.broadcast_to(recips[:, None], (NW, L))
    return k(token_ids, recips, embeds_weight)


def kernel(token_ids, sent_lengths, embeds_weight):
    return _mean_encoder_sc(token_ids, sent_lengths, embeds_weight)


# speedup vs baseline: 13.1974x; 1.2879x over previous
import functools

import jax
import jax.numpy as jnp
from jax import lax
from jax.experimental import pallas as pl
from jax.experimental.pallas import tpu as pltpu
from jax.experimental.pallas import tpu_sc as plsc

L = 16


def _mean_encoder_sc(token_ids, sent_lengths, embeds_weight, *, G=32):
    TOTAL = token_ids.shape[0]
    B = sent_lengths.shape[0]
    E = embeds_weight.shape[1]

    info = plsc.get_sparse_core_info()
    NC, NS = info.num_cores, info.num_subcores
    NW = NC * NS
    TPW = TOTAL // NW
    WPS = NW // B
    NB = TPW // G
    EC = E // L
    HALF = E // WPS

    mesh = plsc.VectorSubcoreMesh(core_axis_name="c", subcore_axis_name="s")

    @functools.partial(
        pl.kernel,
        out_type=jax.ShapeDtypeStruct((B, E), jnp.float32),
        mesh=mesh,
        scratch_types=[
            pltpu.VMEM((TPW,), jnp.int32),
            pltpu.VMEM((G, E), jnp.float32),
            pltpu.VMEM((G, E), jnp.float32),
            pltpu.VMEM((E,), jnp.float32),
            pltpu.VMEM((1, E), jnp.float32),
            pltpu.VMEM((HALF,), jnp.float32),
            pltpu.VMEM((HALF,), jnp.float32),
            pltpu.VMEM((L,), jnp.float32),
            pltpu.VMEM_SHARED((NS, E), jnp.float32),
            pltpu.SemaphoreType.DMA,
            pltpu.SemaphoreType.DMA,
            pltpu.SemaphoreType.DMA,
        ],
    )
    def k(tok_hbm, rec_hbm, w_hbm, out_hbm, idx_v, buf0, buf1, acc_v,
          row1_v, part_v, outrow_v, rec_v, shared, sem0, sem1, sem2):
        c = lax.axis_index("c")
        s = lax.axis_index("s")
        wid = c * NS + s
        base = wid * TPW
        sent = wid // WPS
        halfsel = wid % WPS
        col0 = halfsel * HALF

        pltpu.sync_copy(tok_hbm.at[pl.ds(base, TPW)], idx_v)
        pltpu.sync_copy(rec_hbm.at[wid], rec_v)

        bufs = (buf0, buf1)
        sems = (sem0, sem1)

        def start(g, buf, sem):
            pltpu.make_async_copy(
                w_hbm.at[idx_v.at[pl.ds(g * G, G)]], buf, sem).start()

        def wait(buf, sem):
            pltpu.make_async_copy(
                w_hbm.at[idx_v.at[pl.ds(0, G)]], buf, sem).wait()

        zero = jnp.zeros((L,), jnp.float32)

        def zbody(j, carry):
            acc_v[pl.ds(j * L, L)] = zero
            return carry

        lax.fori_loop(0, EC, zbody, 0)

        start(0, buf0, sem0)
        pltpu.make_async_copy(
            w_hbm.at[idx_v.at[pl.ds(0, 1)]], row1_v, sem2).start()

        def accum(buf):
            def abody(j, carry):
                dsl = pl.ds(j * L, L)
                lanes = [buf[r, dsl] for r in range(4)]
                for r in range(4, G):
                    lanes[r % 4] = lanes[r % 4] + buf[r, dsl]
                v = (lanes[0] + lanes[1]) + (lanes[2] + lanes[3])
                acc_v[dsl] = acc_v[dsl] + v
                return carry
            lax.fori_loop(0, EC, abody, 0)

        def pair_body(i, carry):
            for half in range(2):
                g = i * 2 + half
                p = half
                nxt = g + 1

                @pl.when(nxt < NB)
                def _():
                    start(nxt, bufs[1 - p], sems[1 - p])

                wait(bufs[p], sems[p])
                accum(bufs[p])
            return carry

        lax.fori_loop(0, NB // 2, pair_body, 0)

        pltpu.make_async_copy(
            w_hbm.at[idx_v.at[pl.ds(0, 1)]], row1_v, sem2).wait()
        exm = jnp.where(halfsel == 0, 1.0, 0.0).astype(jnp.float32)

        def sbody(j, carry):
            dsl = pl.ds(j * L, L)
            acc_v[dsl] = acc_v[dsl] - row1_v[0, dsl] * exm
            return carry

        lax.fori_loop(0, EC, sbody, 0)

        pltpu.sync_copy(acc_v, shared.at[s])
        plsc.subcore_barrier()
        partner = jnp.bitwise_xor(s, 1)
        pltpu.sync_copy(shared.at[partner, pl.ds(col0, HALF)], part_v)

        rec = rec_v[...]

        def fbody(j, carry):
            dsl = pl.ds(col0 + j * L, L)
            dso = pl.ds(j * L, L)
            outrow_v[dso] = (acc_v[dsl] + part_v[dso]) * rec
            return carry

        lax.fori_loop(0, HALF // L, fbody, 0)

        pltpu.sync_copy(outrow_v, out_hbm.at[sent, pl.ds(col0, HALF)])

    recips = (1.0 / sent_lengths.astype(jnp.float32))
    recips = jnp.repeat(recips, WPS)
    recips = jnp.broadcast_to(recips[:, None], (NW, L))
    return k(token_ids, recips, embeds_weight)


def kernel(token_ids, sent_lengths, embeds_weight):
    return _mean_encoder_sc(token_ids, sent_lengths, embeds_weight)


# speedup vs baseline: 14.0199x; 1.0623x over previous
import functools

import jax
import jax.numpy as jnp
from jax import lax
from jax.experimental import pallas as pl
from jax.experimental.pallas import tpu as pltpu
from jax.experimental.pallas import tpu_sc as plsc

L = 16


def _mean_encoder_sc(token_ids, sent_lengths, embeds_weight, *, G=32):
    TOTAL = token_ids.shape[0]
    B = sent_lengths.shape[0]
    E = embeds_weight.shape[1]

    info = plsc.get_sparse_core_info()
    NC, NS = info.num_cores, info.num_subcores
    NW = NC * NS
    TPW = TOTAL // NW
    WPS = NW // B
    NB = TPW // G
    EC = E // L
    HALF = E // WPS

    mesh = plsc.VectorSubcoreMesh(core_axis_name="c", subcore_axis_name="s")

    @functools.partial(
        pl.kernel,
        out_type=jax.ShapeDtypeStruct((B, E), jnp.float32),
        mesh=mesh,
        scratch_types=[
            pltpu.VMEM((TPW,), jnp.int32),
            pltpu.VMEM((G, E), jnp.float32),
            pltpu.VMEM((G, E), jnp.float32),
            pltpu.VMEM((E,), jnp.float32),
            pltpu.VMEM((1, E), jnp.float32),
            pltpu.VMEM((HALF,), jnp.float32),
            pltpu.VMEM((HALF,), jnp.float32),
            pltpu.VMEM((L,), jnp.float32),
            pltpu.VMEM_SHARED((NS, E), jnp.float32),
            pltpu.SemaphoreType.DMA,
            pltpu.SemaphoreType.DMA,
            pltpu.SemaphoreType.DMA,
        ],
    )
    def k(tok_hbm, rec_hbm, w_hbm, out_hbm, idx_v, buf0, buf1, acc_v,
          row1_v, part_v, outrow_v, rec_v, shared, sem0, sem1, sem2):
        c = lax.axis_index("c")
        s = lax.axis_index("s")
        wid = c * NS + s
        base = wid * TPW
        sent = wid // WPS
        halfsel = wid % WPS
        col0 = halfsel * HALF

        pltpu.sync_copy(tok_hbm.at[pl.ds(base, TPW)], idx_v)
        pltpu.sync_copy(rec_hbm.at[wid], rec_v)

        bufs = (buf0, buf1)
        sems = (sem0, sem1)

        def start(g, buf, sem):
            pltpu.make_async_copy(
                w_hbm.at[idx_v.at[pl.ds(g * G, G)]], buf, sem).start()

        def wait(buf, sem):
            pltpu.make_async_copy(
                w_hbm.at[idx_v.at[pl.ds(0, G)]], buf, sem).wait()

        zero = jnp.zeros((L,), jnp.float32)

        def zbody(j, carry):
            acc_v[pl.ds(j * L, L)] = zero
            return carry

        lax.fori_loop(0, EC, zbody, 0)

        start(0, buf0, sem0)
        pltpu.make_async_copy(
            w_hbm.at[idx_v.at[pl.ds(0, 1)]], row1_v, sem2).start()

        def accum(buf):
            @plsc.parallel_loop(0, EC, 1, unroll=2)
            def _(j):
                dsl = pl.ds(j * L, L)
                lanes = [buf[r, dsl] for r in range(4)]
                for r in range(4, G):
                    lanes[r % 4] = lanes[r % 4] + buf[r, dsl]
                v = (lanes[0] + lanes[1]) + (lanes[2] + lanes[3])
                acc_v[dsl] = acc_v[dsl] + v

        def pair_body(i, carry):
            for half in range(2):
                g = i * 2 + half
                p = half
                nxt = g + 1

                @pl.when(nxt < NB)
                def _():
                    start(nxt, bufs[1 - p], sems[1 - p])

                wait(bufs[p], sems[p])
                accum(bufs[p])
            return carry

        lax.fori_loop(0, NB // 2, pair_body, 0)

        pltpu.make_async_copy(
            w_hbm.at[idx_v.at[pl.ds(0, 1)]], row1_v, sem2).wait()
        exm = jnp.where(halfsel == 0, 1.0, 0.0).astype(jnp.float32)

        def sbody(j, carry):
            dsl = pl.ds(j * L, L)
            acc_v[dsl] = acc_v[dsl] - row1_v[0, dsl] * exm
            return carry

        lax.fori_loop(0, EC, sbody, 0)

        pltpu.sync_copy(acc_v, shared.at[s])
        plsc.subcore_barrier()
        partner = jnp.bitwise_xor(s, 1)
        pltpu.sync_copy(shared.at[partner, pl.ds(col0, HALF)], part_v)

        rec = rec_v[...]

        def fbody(j, carry):
            dsl = pl.ds(col0 + j * L, L)
            dso = pl.ds(j * L, L)
            outrow_v[dso] = (acc_v[dsl] + part_v[dso]) * rec
            return carry

        lax.fori_loop(0, HALF // L, fbody, 0)

        pltpu.sync_copy(outrow_v, out_hbm.at[sent, pl.ds(col0, HALF)])

    recips = (1.0 / sent_lengths.astype(jnp.float32))
    recips = jnp.repeat(recips, WPS)
    recips = jnp.broadcast_to(recips[:, None], (NW, L))
    return k(token_ids, recips, embeds_weight)


def kernel(token_ids, sent_lengths, embeds_weight):
    return _mean_encoder_sc(token_ids, sent_lengths, embeds_weight)
